# Initial kernel scaffold; baseline (speedup 1.0000x reference)
import jax
import jax.numpy as jnp
from jax.experimental import pallas as pl


def kernel(features, class_labels, centers):
    raise NotImplementedError("write your pallas kernel here")


# speedup vs baseline: 1.9146x; 1.9146x over previous
import functools

import jax
import jax.numpy as jnp
from jax import lax
from jax.experimental import pallas as pl
from jax.experimental.pallas import tpu as pltpu
from jax.experimental.pallas import tpu_sc as plsc

NUM_CLASSES = 1000
D = 128
B = 16384
L = 16
NC, NS = 2, 16
NW = NC * NS
BPW = B // NW
CHUNK = 128
NCHUNK = BPW // CHUNK


_mesh = plsc.VectorSubcoreMesh(core_axis_name="c", subcore_axis_name="s")


@functools.partial(
    pl.kernel,
    mesh=_mesh,
    out_type=jax.ShapeDtypeStruct((NW, L), jnp.float32),
    scratch_types=[
        pltpu.VMEM((BPW,), jnp.int32),
        pltpu.VMEM((CHUNK, D), jnp.float32),
        pltpu.VMEM((CHUNK, D), jnp.float32),
        pltpu.VMEM((L,), jnp.float32),
        pltpu.SemaphoreType.DMA,
    ],
)
def _center_loss_partials(feat_hbm, lbl_hbm, cent_hbm, out_hbm,
                          idx_v, rows_v, feat_v, acc_v, sem):
    wid = lax.axis_index("s") * NC + lax.axis_index("c")
    base = wid * BPW
    pltpu.sync_copy(lbl_hbm.at[pl.ds(base, BPW)], idx_v)

    acc = jnp.zeros((L,), jnp.float32)
    for ch in range(NCHUNK):
        gather = pltpu.async_copy(
            cent_hbm.at[idx_v.at[pl.ds(ch * CHUNK, CHUNK)]], rows_v, sem)
        pltpu.sync_copy(feat_hbm.at[pl.ds(base + ch * CHUNK, CHUNK)], feat_v)
        gather.wait()

        def body(r, a):
            for j in range(D // L):
                f = feat_v[r, pl.ds(j * L, L)]
                c = rows_v[r, pl.ds(j * L, L)]
                d = f - c
                a = a + d * d
            return a

        acc = lax.fori_loop(0, CHUNK, body, acc)

    acc_v[...] = acc * jnp.float32(1.0 / (B * D))
    pltpu.sync_copy(acc_v, out_hbm.at[wid])


def kernel(features, class_labels, centers):
    partials = _center_loss_partials(
        features, class_labels.astype(jnp.int32), centers)
    return jnp.sum(partials)
